# baseline (speedup 1.0000x reference)
import functools

import jax
import jax.numpy as jnp
from jax import lax
from jax.experimental import pallas as pl
from jax.experimental.pallas import tpu as pltpu
from jax.experimental.pallas import tpu_sc as plsc

_NV = 1000001
_B = 16384
_D = 64
_NW = 32
_RNG = 32768
_BLK = 512
_TAILC0 = (_NV // _BLK) * _BLK
_HCH = 2048
_BATCH = 128
_NCTX = _B * 20
_NCN = _B * 6
_PAIR_ROWS = 8448
_DUMP_PAIR = 8192
_CN_ROWS = _NCN + _BATCH
dump_slot = _NCN


def _sload(ref, i):
    return plsc.load_gather(ref, [jnp.full((16,), i, jnp.int32)])[0]


def _hit_walker(sorted_idx, sorted_dst, bv, sbufs, tailv, idxc, dstc,
                rstage, didx, wid, nfull, cb0, table, sems, emit, flush):
    dvecs = [lax.iota(jnp.int32, 16) + 16 * q for q in range(4)]

    def make_hit_body(buf, base_col):
        def hit_body(j, carry):
            ctr, chunk_lo = carry
            need = j >= chunk_lo + (_HCH - 16)
            new_lo = jnp.where(need, j & -8, chunk_lo)

            @pl.when(need)
            def _():
                a0 = pl.multiple_of(j & -8, 8)
                pltpu.sync_copy(sorted_idx.at[pl.ds(a0, _HCH)], idxc)
                pltpu.sync_copy(sorted_dst.at[pl.ds(a0, _HCH)], dstc)

            local = j - new_lo
            c = _sload(idxc, local)
            dst = _sload(dstc, local)
            c_loc = c - base_col
            cvec = jnp.full((16,), c_loc, jnp.int32)
            rq = [plsc.load_gather(buf, [dvecs[q], cvec]) for q in range(4)]
            ctr2 = emit(ctr, rq, dst)
            do_flush = ctr2 == _BATCH

            @pl.when(do_flush)
            def _():
                flush()

            return (jnp.where(do_flush, 0, ctr2), new_lo)

        return hit_body

    def pair_body(i, carry):
        for par in range(2):
            n = i * 2 + par

            @pl.when(n < nfull)
            def _():
                pltpu.make_async_copy(
                    table.at[:, pl.ds(cb0 + n * _BLK, _BLK)],
                    sbufs[par], sems.at[par]).wait()

            e0 = _sload(bv, n)
            e1 = jnp.where(n < nfull, _sload(bv, n + 1), e0)
            carry = lax.fori_loop(e0, e1, make_hit_body(sbufs[par],
                                                        cb0 + n * _BLK),
                                  carry)

            @pl.when(n + 2 < nfull)
            def _():
                pltpu.async_copy(
                    table.at[:, pl.ds(cb0 + (n + 2) * _BLK, _BLK)],
                    sbufs[par], sems.at[par])

        return carry

    for par in range(2):
        @pl.when(par < nfull)
        def _():
            pltpu.async_copy(table.at[:, pl.ds(cb0 + par * _BLK, _BLK)],
                             sbufs[par], sems.at[par])

    carry = lax.fori_loop(0, 32, pair_body, (jnp.int32(0), jnp.int32(-2**30)))
    e_t0 = _sload(bv, nfull)
    e_t1 = _sload(bv, 64)
    carry = lax.fori_loop(e_t0, e_t1, make_hit_body(tailv, _TAILC0), carry)
    return carry[0]


def _make_gc_body(dump_slot):
  def _gc_body(scn_idx, scn_dst, bounds, xt, tailt, rows_out,
               sbuf0, sbuf1, tailv, idxc, dstc, bv, rstage, sidx, sems):
      wid = lax.axis_index("s") * 2 + lax.axis_index("c")
      cb0 = wid * _RNG
      nfull = jnp.clip((_NV - cb0) // _BLK, 0, 64)

      pltpu.sync_copy(tailt, tailv)
      pltpu.sync_copy(bounds.at[pl.ds(pl.multiple_of(wid * 64, 8), 80)], bv)

      lane = lax.iota(jnp.int32, 16)
      zero16 = jnp.zeros((16,), jnp.float32)
      for k in range(8):
          sidx[pl.ds(k * 16, 16)] = jnp.full((16,), dump_slot, jnp.int32)

      def flush():
          pltpu.async_copy(rstage, rows_out.at[sidx], sems.at[2]).wait()

      def emit(ctr, rq, dst):
          for q in range(4):
              rstage[ctr, pl.ds(q * 16, 16)] = rq[q]
              rstage[ctr, pl.ds(64 + q * 16, 16)] = zero16
          plsc.store_scatter(sidx, [jnp.full((16,), ctr, jnp.int32)],
                             jnp.full((16,), dst, jnp.int32), mask=lane == 0)
          return ctr + 1

      ctr = _hit_walker(scn_idx, scn_dst, bv, (sbuf0, sbuf1), tailv, idxc, dstc,
                        rstage, sidx, wid, nfull, cb0, xt, sems, emit, flush)

      def pad_body(r, _):
          @pl.when(r >= ctr)
          def _():
              plsc.store_scatter(sidx, [jnp.full((16,), r, jnp.int32)],
                                 jnp.full((16,), dump_slot, jnp.int32),
                                 mask=lane == 0)
          return 0

      lax.fori_loop(0, _BATCH, pad_body, 0)
      flush()


  return _gc_body


def _k3_body(cn_ref, ctx_ref, out_ref):
    ctx = ctx_ref[...].reshape(256, 20, 128)
    csum = jnp.sum(ctx, axis=1).reshape(256, 1, 128)
    cn = cn_ref[...].reshape(256, 6, 128)
    out_ref[...] = jnp.sum(cn * csum, axis=2)


def kernel(x, center_table, context_table):
    xm = (x + _NV) % _NV
    cn = xm[:, :6].reshape(_NCN)
    cx = xm[:, 6:].reshape(_NCTX)

    def prep(idx_flat, dstdiv):
        iota = jnp.arange(idx_flat.shape[0], dtype=jnp.int32)
        sidx, order = lax.sort((idx_flat, iota), num_keys=1)
        bounds = jnp.searchsorted(sidx, jnp.arange(0, 1048577, _BLK,
                                                   dtype=jnp.int32),
                                  ).astype(jnp.int32)
        bounds = jnp.pad(bounds, (0, 2064 - bounds.shape[0]))
        sidx = jnp.pad(sidx, (0, _HCH))
        dst = jnp.pad(order, (0, _HCH))
        return sidx, dst, bounds

    scx_idx, scx_dst, bounds_cx = prep(cx, 20)
    scn_idx, scn_dst, bounds_cn = prep(cn, 6)

    ctx_t = context_table.T
    cen_t = center_table.T
    tail_cx = jnp.pad(ctx_t[:, _TAILC0:], ((0, 0), (0, 128 - (_NV - _TAILC0))))
    tail_cn = jnp.pad(cen_t[:, _TAILC0:], ((0, 0), (0, 128 - (_NV - _TAILC0))))

    mesh = plsc.VectorSubcoreMesh(core_axis_name="c", subcore_axis_name="s")
    cparams = pltpu.CompilerParams(needs_layout_passes=False)

    def gc_kernel(n_rows, dump_slot):
        return pl.kernel(
            _make_gc_body(dump_slot),
            out_type=jax.ShapeDtypeStruct((n_rows, 128), jnp.float32),
            mesh=mesh,
            compiler_params=cparams,
            scratch_types=[
                pltpu.VMEM((64, _BLK), jnp.float32),
                pltpu.VMEM((64, _BLK), jnp.float32),
                pltpu.VMEM((64, 128), jnp.float32),
                pltpu.VMEM((_HCH,), jnp.int32),
                pltpu.VMEM((_HCH,), jnp.int32),
                pltpu.VMEM((80,), jnp.int32),
                pltpu.VMEM((_BATCH, 128), jnp.float32),
                pltpu.VMEM((_BATCH,), jnp.int32),
                pltpu.SemaphoreType.DMA((3,)),
            ],
        )

    ctx_rows = gc_kernel(_NCTX + _BATCH, _NCTX)(
        scx_idx, scx_dst, bounds_cx, ctx_t, tail_cx)
    cn_rows = gc_kernel(_NCN + _BATCH, _NCN)(
        scn_idx, scn_dst, bounds_cn, cen_t, tail_cn)

    k3 = pl.pallas_call(
        _k3_body,
        out_shape=jax.ShapeDtypeStruct((_B, 6), jnp.float32),
        grid=(_B // 256,),
        in_specs=[
            pl.BlockSpec((1536, 128), lambda i: (i, 0)),
            pl.BlockSpec((5120, 128), lambda i: (i, 0)),
        ],
        out_specs=pl.BlockSpec((256, 6), lambda i: (i, 0)),
    )
    scores = k3(cn_rows, ctx_rows)
    return (scores[:, :1], scores[:, 1:])
